# Initial kernel scaffold; baseline (speedup 1.0000x reference)
import jax
import jax.numpy as jnp
from jax.experimental import pallas as pl


def kernel(x, table, W, b):
    raise NotImplementedError("write your pallas kernel here")


# speedup vs baseline: 1.0160x; 1.0160x over previous
import jax
import jax.numpy as jnp
from jax import lax
from jax.experimental import pallas as pl
from jax.experimental.pallas import tpu as pltpu
from jax.experimental.pallas import tpu_sc as plsc

VOCAB = 1000000
EMBED = 64
MAX_LEN = 20
BATCH = 16384

NUM_CORES = 2
NUM_SUBCORES = 16
LANES = 16
NW = NUM_CORES * NUM_SUBCORES

ROWS_PER_W = BATCH // NW
CHUNK = 32
NCHUNK = ROWS_PER_W // CHUNK
IDX_PER_CHUNK = CHUNK * MAX_LEN
GATHERS = IDX_PER_CHUNK // 128
WGROUPS = EMBED * MAX_LEN // LANES


def _sc_body(x_hbm, table_hbm, w_hbm, b_hbm, out_hbm,
             idx_v, rows_v, w_v, b_v, mat_v, out_v, sem0, sem1):
  wid = lax.axis_index("s") * NUM_CORES + lax.axis_index("c")

  pltpu.sync_copy(
      x_hbm.at[pl.ds(wid * NCHUNK * GATHERS, NCHUNK * GATHERS), :], idx_v)
  pltpu.sync_copy(w_hbm, w_v)
  pltpu.sync_copy(b_hbm, b_v)

  def issue(lc, slot, sem):
    for g in range(GATHERS):
      pltpu.async_copy(
          table_hbm.at[idx_v.at[lc * GATHERS + g]],
          rows_v.at[slot, pl.ds(g * 128, 128), :],
          sem,
      )

  def drain(slot, sem):
    pltpu.make_async_copy(
        table_hbm.at[pl.ds(0, IDX_PER_CHUNK), :], rows_v.at[slot], sem
    ).wait()

  def compute(c_local, slot):
    accs = [jnp.zeros((LANES,), jnp.float32) for _ in range(CHUNK)]

    def body(l, accs):
      out = list(accs)
      for cc in range(4):
        w = w_v[l * 4 + cc, :]
        off = cc * LANES
        for i in range(CHUNK):
          r = rows_v[slot, i * MAX_LEN + l, pl.ds(off, LANES)]
          out[i] = out[i] + r * w
      return tuple(out)

    accs = lax.fori_loop(0, MAX_LEN, body, tuple(accs))

    iota = lax.iota(jnp.int32, LANES)
    for g in range(CHUNK // LANES):
      for k in range(LANES):
        mat_v[k, :] = accs[g * LANES + k]
      ssum = jnp.zeros((LANES,), jnp.float32)
      for j in range(LANES):
        ssum = ssum + plsc.load_gather(
            mat_v, [iota, jnp.full((LANES,), j, jnp.int32)])
      z = ssum + b_v[...]
      res = 1.0 / (1.0 + jnp.exp(-z))
      out_v[pl.ds(c_local * CHUNK + g * LANES, LANES)] = res

  issue(0, 0, sem0)

  def tbody(t, carry):
    issue(2 * t + 1, 1, sem1)
    drain(0, sem0)
    compute(2 * t, 0)

    @pl.when(t < NCHUNK // 2 - 1)
    def _():
      issue(2 * t + 2, 0, sem0)

    drain(1, sem1)
    compute(2 * t + 1, 1)
    return carry

  lax.fori_loop(0, NCHUNK // 2, tbody, 0)

  pltpu.sync_copy(out_v, out_hbm.at[pl.ds(wid * ROWS_PER_W, ROWS_PER_W)])


_mesh = plsc.VectorSubcoreMesh(
    core_axis_name="c", subcore_axis_name="s",
    num_cores=NUM_CORES, num_subcores=NUM_SUBCORES,
)

_sc_call = pl.kernel(
    _sc_body,
    out_type=jax.ShapeDtypeStruct((BATCH,), jnp.float32),
    mesh=_mesh,
    compiler_params=pltpu.CompilerParams(
        needs_layout_passes=False, use_tc_tiling_on_sc=False),
    scratch_types=[
        pltpu.VMEM((NCHUNK * GATHERS, 128), jnp.int32),
        pltpu.VMEM((2, IDX_PER_CHUNK, EMBED), jnp.float32),
        pltpu.VMEM((WGROUPS, LANES), jnp.float32),
        pltpu.VMEM((LANES,), jnp.float32),
        pltpu.VMEM((LANES, LANES), jnp.float32),
        pltpu.VMEM((ROWS_PER_W,), jnp.float32),
        pltpu.SemaphoreType.DMA,
        pltpu.SemaphoreType.DMA,
    ],
)


@jax.jit
def kernel(x, table, W, b):
  x32 = x.astype(jnp.int32).reshape(BATCH * MAX_LEN // 128, 128)
  w2 = W.astype(jnp.float32).reshape(WGROUPS, LANES)
  b16 = jnp.broadcast_to(b.astype(jnp.float32).reshape(()), (LANES,))
  out = _sc_call(x32, table, w2, b16)
  return out.reshape(BATCH, 1)


# speedup vs baseline: 1.3937x; 1.3718x over previous
import jax
import jax.numpy as jnp
from jax import lax
from jax.experimental import pallas as pl
from jax.experimental.pallas import tpu as pltpu
from jax.experimental.pallas import tpu_sc as plsc

VOCAB = 1000000
EMBED = 64
MAX_LEN = 20
BATCH = 16384

NUM_CORES = 2
NUM_SUBCORES = 16
LANES = 16
NW = NUM_CORES * NUM_SUBCORES

ROWS_PER_W = BATCH // NW
QCHUNKS = ROWS_PER_W // 128

TC_BLOCK = 2048


def _tc_body(w_ref, t_ref, *out_refs):
  tblk = t_ref[...]
  w = w_ref[...]
  m = lax.dot_general(
      w, tblk, (((0,), (0,)), ((), ())),
      preferred_element_type=jnp.float32,
      precision=lax.Precision.HIGHEST)
  for l in range(MAX_LEN):
    out_refs[l][...] = m[l, :]


_tc_call = pl.pallas_call(
    _tc_body,
    grid=(pl.cdiv(VOCAB, TC_BLOCK),),
    in_specs=[
        pl.BlockSpec((EMBED, MAX_LEN), lambda j: (0, 0)),
        pl.BlockSpec((EMBED, TC_BLOCK), lambda j: (0, j)),
    ],
    out_specs=[pl.BlockSpec((TC_BLOCK,), lambda j: (j,))] * MAX_LEN,
    out_shape=[jax.ShapeDtypeStruct((VOCAB,), jnp.float32)] * MAX_LEN,
)


def _sc_body(xt_hbm, b_hbm, *rest):
  proj_hbm = rest[:MAX_LEN]
  out_hbm = rest[MAX_LEN]
  idx_v, g_v, b_v, out_v, sem = rest[MAX_LEN + 1:]

  wid = lax.axis_index("s") * NUM_CORES + lax.axis_index("c")
  base = wid * ROWS_PER_W

  pltpu.sync_copy(b_hbm, b_v)
  pltpu.sync_copy(xt_hbm.at[:, pl.ds(base, ROWS_PER_W)], idx_v)

  for l in range(MAX_LEN):
    for q in range(QCHUNKS):
      pltpu.async_copy(
          proj_hbm[l].at[idx_v.at[l, pl.ds(q * 128, 128)]],
          g_v.at[l, pl.ds(q * 128, 128)],
          sem,
      )
  for l in range(MAX_LEN):
    pltpu.make_async_copy(
        proj_hbm[l].at[pl.ds(0, ROWS_PER_W)], g_v.at[l], sem
    ).wait()

  for g in range(ROWS_PER_W // LANES):
    acc = g_v[0, pl.ds(g * LANES, LANES)]
    for l in range(1, MAX_LEN):
      acc = acc + g_v[l, pl.ds(g * LANES, LANES)]
    z = acc + b_v[...]
    out_v[pl.ds(g * LANES, LANES)] = 1.0 / (1.0 + jnp.exp(-z))

  pltpu.sync_copy(out_v, out_hbm.at[pl.ds(base, ROWS_PER_W)])


_mesh = plsc.VectorSubcoreMesh(
    core_axis_name="c", subcore_axis_name="s",
    num_cores=NUM_CORES, num_subcores=NUM_SUBCORES,
)

_sc_call = pl.kernel(
    _sc_body,
    out_type=jax.ShapeDtypeStruct((BATCH,), jnp.float32),
    mesh=_mesh,
    compiler_params=pltpu.CompilerParams(
        needs_layout_passes=False, use_tc_tiling_on_sc=False),
    scratch_types=[
        pltpu.VMEM((MAX_LEN, ROWS_PER_W), jnp.int32),
        pltpu.VMEM((MAX_LEN, ROWS_PER_W), jnp.float32),
        pltpu.VMEM((LANES,), jnp.float32),
        pltpu.VMEM((ROWS_PER_W,), jnp.float32),
        pltpu.SemaphoreType.DMA,
    ],
)


@jax.jit
def kernel(x, table, W, b):
  xt = x.astype(jnp.int32).T
  tt = table.T
  w64 = W.astype(jnp.float32).reshape(MAX_LEN, EMBED).T
  b16 = jnp.broadcast_to(b.astype(jnp.float32).reshape(()), (LANES,))
  projs = _tc_call(w64, tt)
  out = _sc_call(xt, b16, *projs)
  return out.reshape(BATCH, 1)


# speedup vs baseline: 2.9124x; 2.0897x over previous
import jax
import jax.numpy as jnp
from jax import lax
from jax.experimental import pallas as pl
from jax.experimental.pallas import tpu as pltpu
from jax.experimental.pallas import tpu_sc as plsc

VOCAB = 1000000
EMBED = 64
MAX_LEN = 20
BATCH = 16384

NUM_CORES = 2
NUM_SUBCORES = 16
LANES = 16
NW = NUM_CORES * NUM_SUBCORES

ROWS_PER_W = BATCH // NW
QCHUNKS = ROWS_PER_W // 128

TC_BLOCK = 8192


def _tc_body(w_ref, t_ref, *out_refs):
  tblk = t_ref[...]
  w = w_ref[...]
  m = lax.dot_general(
      w, tblk, (((0,), (0,)), ((), ())),
      preferred_element_type=jnp.float32,
      precision=lax.Precision.HIGHEST)
  for l in range(MAX_LEN):
    out_refs[l][...] = m[l, :]


_tc_call = pl.pallas_call(
    _tc_body,
    grid=(pl.cdiv(VOCAB, TC_BLOCK),),
    in_specs=[
        pl.BlockSpec((EMBED, MAX_LEN), lambda j: (0, 0)),
        pl.BlockSpec((EMBED, TC_BLOCK), lambda j: (0, j)),
    ],
    out_specs=[pl.BlockSpec((TC_BLOCK,), lambda j: (j,))] * MAX_LEN,
    out_shape=[jax.ShapeDtypeStruct((VOCAB,), jnp.float32)] * MAX_LEN,
)


def _sc_body(xt_hbm, b_hbm, *rest):
  proj_hbm = rest[:MAX_LEN]
  out_hbm = rest[MAX_LEN]
  idx_v, g_v, b_v, out_v, sem = rest[MAX_LEN + 1:]

  wid = lax.axis_index("s") * NUM_CORES + lax.axis_index("c")
  base = wid * ROWS_PER_W

  pltpu.sync_copy(b_hbm, b_v)
  pltpu.sync_copy(xt_hbm.at[:, pl.ds(base, ROWS_PER_W)], idx_v)

  for l in range(MAX_LEN):
    for q in range(QCHUNKS):
      pltpu.async_copy(
          proj_hbm[l].at[idx_v.at[l, pl.ds(q * 128, 128)]],
          g_v.at[l, pl.ds(q * 128, 128)],
          sem,
      )
  for l in range(MAX_LEN):
    pltpu.make_async_copy(
        proj_hbm[l].at[pl.ds(0, ROWS_PER_W)], g_v.at[l], sem
    ).wait()

  for g in range(ROWS_PER_W // LANES):
    acc = g_v[0, pl.ds(g * LANES, LANES)]
    for l in range(1, MAX_LEN):
      acc = acc + g_v[l, pl.ds(g * LANES, LANES)]
    z = acc + b_v[...]
    out_v[pl.ds(g * LANES, LANES)] = 1.0 / (1.0 + jnp.exp(-z))

  pltpu.sync_copy(out_v, out_hbm.at[pl.ds(base, ROWS_PER_W)])


_mesh = plsc.VectorSubcoreMesh(
    core_axis_name="c", subcore_axis_name="s",
    num_cores=NUM_CORES, num_subcores=NUM_SUBCORES,
)

_sc_call = pl.kernel(
    _sc_body,
    out_type=jax.ShapeDtypeStruct((BATCH,), jnp.float32),
    mesh=_mesh,
    compiler_params=pltpu.CompilerParams(
        needs_layout_passes=False, use_tc_tiling_on_sc=False),
    scratch_types=[
        pltpu.VMEM((MAX_LEN, ROWS_PER_W), jnp.int32),
        pltpu.VMEM((MAX_LEN, ROWS_PER_W), jnp.float32),
        pltpu.VMEM((LANES,), jnp.float32),
        pltpu.VMEM((ROWS_PER_W,), jnp.float32),
        pltpu.SemaphoreType.DMA,
    ],
)


@jax.jit
def kernel(x, table, W, b):
  xt = x.astype(jnp.int32).T
  tt = table.T
  w64 = W.astype(jnp.float32).reshape(MAX_LEN, EMBED).T
  b16 = jnp.broadcast_to(b.astype(jnp.float32).reshape(()), (LANES,))
  projs = _tc_call(w64, tt)
  out = _sc_call(xt, b16, *projs)
  return out.reshape(BATCH, 1)


# speedup vs baseline: 4.0038x; 1.3747x over previous
import jax
import jax.numpy as jnp
from jax import lax
from jax.experimental import pallas as pl
from jax.experimental.pallas import tpu as pltpu
from jax.experimental.pallas import tpu_sc as plsc

VOCAB = 1000000
EMBED = 64
MAX_LEN = 20
BATCH = 16384

NUM_CORES = 2
NUM_SUBCORES = 16
LANES = 16
NW = NUM_CORES * NUM_SUBCORES

ROWS_PER_W = BATCH // NW
QCHUNKS = ROWS_PER_W // 128

TC_BLOCK = 32768


def _tc_body(w_ref, t_ref, *out_refs):
  tblk = t_ref[...]
  w = w_ref[...]
  m = lax.dot_general(
      w, tblk, (((0,), (0,)), ((), ())),
      preferred_element_type=jnp.float32,
      precision=lax.Precision.HIGHEST)
  for l in range(MAX_LEN):
    out_refs[l][...] = m[l, :]


_tc_call = pl.pallas_call(
    _tc_body,
    grid=(pl.cdiv(VOCAB, TC_BLOCK),),
    in_specs=[
        pl.BlockSpec((EMBED, MAX_LEN), lambda j: (0, 0)),
        pl.BlockSpec((EMBED, TC_BLOCK), lambda j: (0, j)),
    ],
    out_specs=[pl.BlockSpec((TC_BLOCK,), lambda j: (j,))] * MAX_LEN,
    out_shape=[jax.ShapeDtypeStruct((VOCAB,), jnp.float32)] * MAX_LEN,
)


def _sc_body(xt_hbm, b_hbm, *rest):
  proj_hbm = rest[:MAX_LEN]
  out_hbm = rest[MAX_LEN]
  idx_v, g_v, b_v, out_v, sem = rest[MAX_LEN + 1:]

  wid = lax.axis_index("s") * NUM_CORES + lax.axis_index("c")
  base = wid * ROWS_PER_W

  pltpu.sync_copy(b_hbm, b_v)
  pltpu.sync_copy(xt_hbm.at[:, pl.ds(base, ROWS_PER_W)], idx_v)

  for l in range(MAX_LEN):
    for q in range(QCHUNKS):
      pltpu.async_copy(
          proj_hbm[l].at[idx_v.at[l, pl.ds(q * 128, 128)]],
          g_v.at[l, pl.ds(q * 128, 128)],
          sem,
      )
  for l in range(MAX_LEN):
    pltpu.make_async_copy(
        proj_hbm[l].at[pl.ds(0, ROWS_PER_W)], g_v.at[l], sem
    ).wait()

  for g in range(ROWS_PER_W // LANES):
    acc = g_v[0, pl.ds(g * LANES, LANES)]
    for l in range(1, MAX_LEN):
      acc = acc + g_v[l, pl.ds(g * LANES, LANES)]
    z = acc + b_v[...]
    out_v[pl.ds(g * LANES, LANES)] = 1.0 / (1.0 + jnp.exp(-z))

  pltpu.sync_copy(out_v, out_hbm.at[pl.ds(base, ROWS_PER_W)])


_mesh = plsc.VectorSubcoreMesh(
    core_axis_name="c", subcore_axis_name="s",
    num_cores=NUM_CORES, num_subcores=NUM_SUBCORES,
)

_sc_call = pl.kernel(
    _sc_body,
    out_type=jax.ShapeDtypeStruct((BATCH,), jnp.float32),
    mesh=_mesh,
    compiler_params=pltpu.CompilerParams(
        needs_layout_passes=False, use_tc_tiling_on_sc=False),
    scratch_types=[
        pltpu.VMEM((MAX_LEN, ROWS_PER_W), jnp.int32),
        pltpu.VMEM((MAX_LEN, ROWS_PER_W), jnp.float32),
        pltpu.VMEM((LANES,), jnp.float32),
        pltpu.VMEM((ROWS_PER_W,), jnp.float32),
        pltpu.SemaphoreType.DMA,
    ],
)


@jax.jit
def kernel(x, table, W, b):
  xt = x.astype(jnp.int32).T
  tt = table.T
  w64 = W.astype(jnp.float32).reshape(MAX_LEN, EMBED).T
  b16 = jnp.broadcast_to(b.astype(jnp.float32).reshape(()), (LANES,))
  projs = _tc_call(w64, tt)
  out = _sc_call(xt, b16, *projs)
  return out.reshape(BATCH, 1)


# speedup vs baseline: 4.1451x; 1.0353x over previous
import jax
import jax.numpy as jnp
from jax import lax
from jax.experimental import pallas as pl
from jax.experimental.pallas import tpu as pltpu
from jax.experimental.pallas import tpu_sc as plsc

VOCAB = 1000000
EMBED = 64
MAX_LEN = 20
BATCH = 16384

NUM_CORES = 2
NUM_SUBCORES = 16
LANES = 16
NW = NUM_CORES * NUM_SUBCORES

ROWS_PER_W = BATCH // NW
QCHUNKS = ROWS_PER_W // 128

TC_BLOCK = 65536


def _tc_body(w_ref, t_ref, *out_refs):
  tblk = t_ref[...]
  w = w_ref[...]
  m = lax.dot_general(
      w, tblk, (((0,), (0,)), ((), ())),
      preferred_element_type=jnp.float32,
      precision=lax.Precision.HIGHEST)
  for l in range(MAX_LEN):
    out_refs[l][...] = m[l, :]


_tc_call = pl.pallas_call(
    _tc_body,
    grid=(pl.cdiv(VOCAB, TC_BLOCK),),
    in_specs=[
        pl.BlockSpec((EMBED, MAX_LEN), lambda j: (0, 0)),
        pl.BlockSpec((EMBED, TC_BLOCK), lambda j: (0, j)),
    ],
    out_specs=[pl.BlockSpec((TC_BLOCK,), lambda j: (j,))] * MAX_LEN,
    out_shape=[jax.ShapeDtypeStruct((VOCAB,), jnp.float32)] * MAX_LEN,
)


def _sc_body(xt_hbm, b_hbm, *rest):
  proj_hbm = rest[:MAX_LEN]
  out_hbm = rest[MAX_LEN]
  idx_v, g_v, b_v, out_v, sem = rest[MAX_LEN + 1:]

  wid = lax.axis_index("s") * NUM_CORES + lax.axis_index("c")
  base = wid * ROWS_PER_W

  pltpu.sync_copy(b_hbm, b_v)
  pltpu.sync_copy(xt_hbm.at[:, pl.ds(base, ROWS_PER_W)], idx_v)

  for l in range(MAX_LEN):
    for q in range(QCHUNKS):
      pltpu.async_copy(
          proj_hbm[l].at[idx_v.at[l, pl.ds(q * 128, 128)]],
          g_v.at[l, pl.ds(q * 128, 128)],
          sem,
      )
  for l in range(MAX_LEN):
    pltpu.make_async_copy(
        proj_hbm[l].at[pl.ds(0, ROWS_PER_W)], g_v.at[l], sem
    ).wait()

  for g in range(ROWS_PER_W // LANES):
    acc = g_v[0, pl.ds(g * LANES, LANES)]
    for l in range(1, MAX_LEN):
      acc = acc + g_v[l, pl.ds(g * LANES, LANES)]
    z = acc + b_v[...]
    out_v[pl.ds(g * LANES, LANES)] = 1.0 / (1.0 + jnp.exp(-z))

  pltpu.sync_copy(out_v, out_hbm.at[pl.ds(base, ROWS_PER_W)])


_mesh = plsc.VectorSubcoreMesh(
    core_axis_name="c", subcore_axis_name="s",
    num_cores=NUM_CORES, num_subcores=NUM_SUBCORES,
)

_sc_call = pl.kernel(
    _sc_body,
    out_type=jax.ShapeDtypeStruct((BATCH,), jnp.float32),
    mesh=_mesh,
    compiler_params=pltpu.CompilerParams(
        needs_layout_passes=False, use_tc_tiling_on_sc=False),
    scratch_types=[
        pltpu.VMEM((MAX_LEN, ROWS_PER_W), jnp.int32),
        pltpu.VMEM((MAX_LEN, ROWS_PER_W), jnp.float32),
        pltpu.VMEM((LANES,), jnp.float32),
        pltpu.VMEM((ROWS_PER_W,), jnp.float32),
        pltpu.SemaphoreType.DMA,
    ],
)


@jax.jit
def kernel(x, table, W, b):
  xt = x.astype(jnp.int32).T
  tt = table.T
  w64 = W.astype(jnp.float32).reshape(MAX_LEN, EMBED).T
  b16 = jnp.broadcast_to(b.astype(jnp.float32).reshape(()), (LANES,))
  projs = _tc_call(w64, tt)
  out = _sc_call(xt, b16, *projs)
  return out.reshape(BATCH, 1)


# speedup vs baseline: 4.8611x; 1.1727x over previous
import jax
import jax.numpy as jnp
from jax import lax
from jax.experimental import pallas as pl
from jax.experimental.pallas import tpu as pltpu
from jax.experimental.pallas import tpu_sc as plsc

VOCAB = 1000000
EMBED = 64
MAX_LEN = 20
BATCH = 16384

NUM_CORES = 2
NUM_SUBCORES = 16
LANES = 16
NW = NUM_CORES * NUM_SUBCORES

ROWS_PER_W = BATCH // NW
QCHUNKS = ROWS_PER_W // 128

TC_BLOCK = 65536


def _tc_body(w_ref, t_ref, *out_refs):
  tblk = t_ref[...]
  w = w_ref[...]
  m = lax.dot_general(
      w, tblk, (((0,), (0,)), ((), ())),
      preferred_element_type=jnp.float32,
      precision=lax.Precision.DEFAULT)
  for l in range(MAX_LEN):
    out_refs[l][...] = m[l, :] + m[MAX_LEN + l, :]


_tc_call = pl.pallas_call(
    _tc_body,
    grid=(pl.cdiv(VOCAB, TC_BLOCK),),
    in_specs=[
        pl.BlockSpec((EMBED, 2 * MAX_LEN), lambda j: (0, 0)),
        pl.BlockSpec((EMBED, TC_BLOCK), lambda j: (0, j)),
    ],
    out_specs=[pl.BlockSpec((TC_BLOCK,), lambda j: (j,))] * MAX_LEN,
    out_shape=[jax.ShapeDtypeStruct((VOCAB,), jnp.float32)] * MAX_LEN,
)


def _sc_body(xt_hbm, b_hbm, *rest):
  proj_hbm = rest[:MAX_LEN]
  out_hbm = rest[MAX_LEN]
  idx_v, g_v, b_v, out_v, sem = rest[MAX_LEN + 1:]

  wid = lax.axis_index("s") * NUM_CORES + lax.axis_index("c")
  base = wid * ROWS_PER_W

  pltpu.sync_copy(b_hbm, b_v)
  pltpu.sync_copy(xt_hbm.at[:, pl.ds(base, ROWS_PER_W)], idx_v)

  for l in range(MAX_LEN):
    for q in range(QCHUNKS):
      pltpu.async_copy(
          proj_hbm[l].at[idx_v.at[l, pl.ds(q * 128, 128)]],
          g_v.at[l, pl.ds(q * 128, 128)],
          sem,
      )
  for l in range(MAX_LEN):
    pltpu.make_async_copy(
        proj_hbm[l].at[pl.ds(0, ROWS_PER_W)], g_v.at[l], sem
    ).wait()

  for g in range(ROWS_PER_W // LANES):
    acc = g_v[0, pl.ds(g * LANES, LANES)]
    for l in range(1, MAX_LEN):
      acc = acc + g_v[l, pl.ds(g * LANES, LANES)]
    z = acc + b_v[...]
    out_v[pl.ds(g * LANES, LANES)] = 1.0 / (1.0 + jnp.exp(-z))

  pltpu.sync_copy(out_v, out_hbm.at[pl.ds(base, ROWS_PER_W)])


_mesh = plsc.VectorSubcoreMesh(
    core_axis_name="c", subcore_axis_name="s",
    num_cores=NUM_CORES, num_subcores=NUM_SUBCORES,
)

_sc_call = pl.kernel(
    _sc_body,
    out_type=jax.ShapeDtypeStruct((BATCH,), jnp.float32),
    mesh=_mesh,
    compiler_params=pltpu.CompilerParams(
        needs_layout_passes=False, use_tc_tiling_on_sc=False),
    scratch_types=[
        pltpu.VMEM((MAX_LEN, ROWS_PER_W), jnp.int32),
        pltpu.VMEM((MAX_LEN, ROWS_PER_W), jnp.float32),
        pltpu.VMEM((LANES,), jnp.float32),
        pltpu.VMEM((ROWS_PER_W,), jnp.float32),
        pltpu.SemaphoreType.DMA,
    ],
)


@jax.jit
def kernel(x, table, W, b):
  xt = x.astype(jnp.int32).T
  tt = table.T
  w64 = W.astype(jnp.float32).reshape(MAX_LEN, EMBED).T
  w_hi = w64.astype(jnp.bfloat16).astype(jnp.float32)
  wstack = jnp.concatenate([w_hi, w64 - w_hi], axis=1)
  b16 = jnp.broadcast_to(b.astype(jnp.float32).reshape(()), (LANES,))
  projs = _tc_call(wstack, tt)
  out = _sc_call(xt, b16, *projs)
  return out.reshape(BATCH, 1)
